# baseline (speedup 1.0000x reference)
import functools
import math

import jax
import jax.numpy as jnp
from jax import lax
from jax.experimental import pallas as pl
from jax.experimental.pallas import tpu as pltpu
from jax.experimental.pallas import tpu_sc as plsc

D_MODEL = 128
LANES = 16
NUM_CORES = 2
NUM_SUBCORES = 16
NUM_WORKERS = NUM_CORES * NUM_SUBCORES
SCALE = math.sqrt(D_MODEL)
NSTAGE = 4
LEAD = 2
JX = 4


@functools.partial(jax.jit, static_argnames=("b0", "b1"))
def _lookup(idx, table, b0, b1):
    mesh = plsc.VectorSubcoreMesh(core_axis_name="c", subcore_axis_name="s")
    xr = b0 // NUM_WORKERS
    nchunks = xr // 2
    njumbo = xr // JX
    cw = idx.shape[2]

    @functools.partial(
        pl.kernel,
        mesh=mesh,
        out_type=jax.ShapeDtypeStruct((b0, b1, D_MODEL), jnp.float32),
        scratch_types=[
            pltpu.VMEM((nchunks, cw), jnp.int32),
            pltpu.VMEM((NSTAGE, cw, D_MODEL), jnp.float32),
            pltpu.VMEM((2, JX, b1, D_MODEL), jnp.float32),
            pltpu.SemaphoreType.DMA((NSTAGE,)),
            pltpu.SemaphoreType.DMA((2,)),
        ],
    )
    def k(idx_hbm, table_hbm, out_hbm, idx_v, stag_v, jum_v, gsem, osem):
        cid = lax.axis_index("c")
        sid = lax.axis_index("s")
        wid = sid * NUM_CORES + cid
        x_base = wid * xr
        pltpu.sync_copy(idx_hbm.at[wid], idx_v)

        def start_gather(j, sb):
            pltpu.async_copy(table_hbm.at[idx_v.at[j]], stag_v.at[sb], gsem.at[sb])

        def wait_gather(j, sb):
            pltpu.make_async_copy(
                table_hbm.at[idx_v.at[j]], stag_v.at[sb], gsem.at[sb]
            ).wait()

        def start_scatter(q, qb):
            pltpu.async_copy(
                jum_v.at[qb], out_hbm.at[pl.ds(x_base + q * JX, JX)], osem.at[qb]
            )

        def wait_scatter(qb):
            pltpu.make_async_copy(
                jum_v.at[qb], out_hbm.at[pl.ds(0, JX)], osem.at[qb]
            ).wait()

        for j in range(LEAD):
            start_gather(j, j)

        def outer(i, carry):
            j0 = i * 4
            q0 = i * 2
            for qoff in range(2):
                qb = qoff
                q = q0 + qoff
                for p in range(2):
                    jo = 2 * qoff + p
                    j = j0 + jo
                    sb = jo % NSTAGE

                    if p == 0:

                        @pl.when(q >= 2)
                        def _():
                            wait_scatter(qb)

                    @pl.when(j + LEAD < nchunks)
                    def _():
                        start_gather(j + LEAD, (jo + LEAD) % NSTAGE)

                    wait_gather(j, sb)

                    def row_body(r, c2):
                        for c in range(D_MODEL // LANES):
                            sl = pl.ds(c * LANES, LANES)
                            jum_v[qb, 2 * p, r, sl] = stag_v[sb, r, sl] * SCALE
                            jum_v[qb, 2 * p + 1, r, sl] = (
                                stag_v[sb, b1 + r, sl] * SCALE
                            )
                        return c2

                    lax.fori_loop(0, b1, row_body, 0, unroll=2)

                start_scatter(q, qb)
            return carry

        lax.fori_loop(0, nchunks // 4, outer, 0)

        for qb in range(2):
            wait_scatter(qb)

    return k(idx, table)


def kernel(x, table):
    b0, b1 = x.shape
    idx = x.astype(jnp.int32).reshape(b0 // 2, 2 * b1)
    cw = -(-(2 * b1) // 8) * 8
    if cw != 2 * b1:
        idx = jnp.pad(idx, ((0, 0), (0, cw - 2 * b1)))
    idx = idx.reshape(NUM_WORKERS, b0 // (2 * NUM_WORKERS), cw)
    return _lookup(idx, table, b0, b1)


# speedup vs baseline: 3.1259x; 3.1259x over previous
import functools
import math

import jax
import jax.numpy as jnp
from jax import lax
from jax.experimental import pallas as pl
from jax.experimental.pallas import tpu as pltpu
from jax.experimental.pallas import tpu_sc as plsc

D_MODEL = 128
LANES = 16
NUM_CORES = 2
NUM_SUBCORES = 16
NUM_WORKERS = NUM_CORES * NUM_SUBCORES
SCALE = math.sqrt(D_MODEL)
NSTAGE = 4
LEAD = 2
JX = 4


@functools.partial(jax.jit, static_argnames=("b0", "b1"))
def _lookup(idx, table, b0, b1):
    mesh = plsc.VectorSubcoreMesh(core_axis_name="c", subcore_axis_name="s")
    xr = b0 // NUM_WORKERS
    nchunks = xr // 2
    njumbo = xr // JX
    cw = idx.shape[2]

    @functools.partial(
        pl.kernel,
        mesh=mesh,
        out_type=jax.ShapeDtypeStruct((b0, b1, D_MODEL), jnp.float32),
        scratch_types=[
            pltpu.VMEM((nchunks, cw), jnp.int32),
            pltpu.VMEM((NSTAGE, cw, D_MODEL), jnp.float32),
            pltpu.VMEM((2, JX, b1, D_MODEL), jnp.float32),
            pltpu.SemaphoreType.DMA((NSTAGE,)),
            pltpu.SemaphoreType.DMA((2,)),
        ],
    )
    def k(idx_hbm, table_hbm, out_hbm, idx_v, stag_v, jum_v, gsem, osem):
        cid = lax.axis_index("c")
        sid = lax.axis_index("s")
        wid = sid * NUM_CORES + cid
        x_base = wid * xr
        pltpu.sync_copy(idx_hbm.at[wid], idx_v)

        def start_gather(j, sb):
            pltpu.async_copy(table_hbm.at[idx_v.at[j]], stag_v.at[sb], gsem.at[sb])

        def wait_gather(j, sb):
            pltpu.make_async_copy(
                table_hbm.at[idx_v.at[j]], stag_v.at[sb], gsem.at[sb]
            ).wait()

        def start_scatter(q, qb):
            pltpu.async_copy(
                jum_v.at[qb], out_hbm.at[pl.ds(x_base + q * JX, JX)], osem.at[qb]
            )

        def wait_scatter(qb):
            pltpu.make_async_copy(
                jum_v.at[qb], out_hbm.at[pl.ds(0, JX)], osem.at[qb]
            ).wait()

        for j in range(LEAD):
            start_gather(j, j)

        def outer(i, carry):
            j0 = i * 4
            q0 = i * 2
            for qoff in range(2):
                qb = qoff
                q = q0 + qoff
                for p in range(2):
                    jo = 2 * qoff + p
                    j = j0 + jo
                    sb = jo % NSTAGE

                    if p == 0:

                        @pl.when(q >= 2)
                        def _():
                            wait_scatter(qb)

                    @pl.when(j + LEAD < nchunks)
                    def _():
                        start_gather(j + LEAD, (jo + LEAD) % NSTAGE)

                    wait_gather(j, sb)

                    def row_body(r, c2):
                        for c in range(D_MODEL // LANES):
                            sl = pl.ds(c * LANES, LANES)
                            jum_v[qb, 2 * p, r, sl] = stag_v[sb, r, sl] * SCALE
                            jum_v[qb, 2 * p + 1, r, sl] = (
                                stag_v[sb, b1 + r, sl] * SCALE
                            )
                        return c2

                    lax.fori_loop(0, b1, row_body, 0, unroll=2)

                start_scatter(q, qb)
            return carry

        lax.fori_loop(0, nchunks // 4, outer, 0)

        for qb in range(2):
            wait_scatter(qb)

    return k(idx, table)


def kernel(x, table):
    b0, b1 = x.shape
    idx = x.astype(jnp.int32).reshape(b0 // 2, 2 * b1)
    cw = -(-(2 * b1) // 8) * 8
    if cw != 2 * b1:
        npad = cw - 2 * b1
        fill = (
            jnp.arange((b0 // 2) * npad, dtype=jnp.int32).reshape(b0 // 2, npad)
            * 9973 % jnp.int32(table.shape[0])
        )
        idx = jnp.concatenate([idx, fill], axis=1)
    idx = idx.reshape(NUM_WORKERS, b0 // (2 * NUM_WORKERS), cw)
    return _lookup(idx, table, b0, b1)


# speedup vs baseline: 3.2554x; 1.0414x over previous
import functools
import math

import jax
import jax.numpy as jnp
from jax import lax
from jax.experimental import pallas as pl
from jax.experimental.pallas import tpu as pltpu
from jax.experimental.pallas import tpu_sc as plsc

D_MODEL = 128
LANES = 16
NUM_CORES = 2
NUM_SUBCORES = 16
NUM_WORKERS = NUM_CORES * NUM_SUBCORES
SCALE = math.sqrt(D_MODEL)
NSTAGE = 4
LEAD = 3
JX = 4


@functools.partial(jax.jit, static_argnames=("b0", "b1"))
def _lookup(idx, table, b0, b1):
    mesh = plsc.VectorSubcoreMesh(core_axis_name="c", subcore_axis_name="s")
    xr = b0 // NUM_WORKERS
    nchunks = xr // 2
    njumbo = xr // JX
    cw = idx.shape[2]

    @functools.partial(
        pl.kernel,
        mesh=mesh,
        out_type=jax.ShapeDtypeStruct((b0, b1, D_MODEL), jnp.float32),
        scratch_types=[
            pltpu.VMEM((nchunks, cw), jnp.int32),
            pltpu.VMEM((NSTAGE, cw, D_MODEL), jnp.float32),
            pltpu.VMEM((2, JX, b1, D_MODEL), jnp.float32),
            pltpu.SemaphoreType.DMA((NSTAGE,)),
            pltpu.SemaphoreType.DMA((2,)),
        ],
    )
    def k(idx_hbm, table_hbm, out_hbm, idx_v, stag_v, jum_v, gsem, osem):
        cid = lax.axis_index("c")
        sid = lax.axis_index("s")
        wid = sid * NUM_CORES + cid
        x_base = wid * xr
        pltpu.sync_copy(idx_hbm.at[wid], idx_v)

        def start_gather(j, sb):
            pltpu.async_copy(table_hbm.at[idx_v.at[j]], stag_v.at[sb], gsem.at[sb])

        def wait_gather(j, sb):
            pltpu.make_async_copy(
                table_hbm.at[idx_v.at[j]], stag_v.at[sb], gsem.at[sb]
            ).wait()

        def start_scatter(q, qb):
            pltpu.async_copy(
                jum_v.at[qb], out_hbm.at[pl.ds(x_base + q * JX, JX)], osem.at[qb]
            )

        def wait_scatter(qb):
            pltpu.make_async_copy(
                jum_v.at[qb], out_hbm.at[pl.ds(0, JX)], osem.at[qb]
            ).wait()

        for j in range(LEAD):
            start_gather(j, j)

        def outer(i, carry):
            j0 = i * 4
            q0 = i * 2
            for qoff in range(2):
                qb = qoff
                q = q0 + qoff
                for p in range(2):
                    jo = 2 * qoff + p
                    j = j0 + jo
                    sb = jo % NSTAGE

                    if p == 0:

                        @pl.when(q >= 2)
                        def _():
                            wait_scatter(qb)

                    @pl.when(j + LEAD < nchunks)
                    def _():
                        start_gather(j + LEAD, (jo + LEAD) % NSTAGE)

                    wait_gather(j, sb)

                    def row_body(r, c2):
                        for c in range(D_MODEL // LANES):
                            sl = pl.ds(c * LANES, LANES)
                            jum_v[qb, 2 * p, r, sl] = stag_v[sb, r, sl] * SCALE
                            jum_v[qb, 2 * p + 1, r, sl] = (
                                stag_v[sb, b1 + r, sl] * SCALE
                            )
                        return c2

                    lax.fori_loop(0, b1, row_body, 0, unroll=2)

                start_scatter(q, qb)
            return carry

        lax.fori_loop(0, nchunks // 4, outer, 0)

        for qb in range(2):
            wait_scatter(qb)

    return k(idx, table)


def kernel(x, table):
    b0, b1 = x.shape
    idx = x.astype(jnp.int32).reshape(b0 // 2, 2 * b1)
    cw = -(-(2 * b1) // 8) * 8
    if cw != 2 * b1:
        npad = cw - 2 * b1
        fill = (
            jnp.arange((b0 // 2) * npad, dtype=jnp.int32).reshape(b0 // 2, npad)
            * 9973 % jnp.int32(table.shape[0])
        )
        idx = jnp.concatenate([idx, fill], axis=1)
    idx = idx.reshape(NUM_WORKERS, b0 // (2 * NUM_WORKERS), cw)
    return _lookup(idx, table, b0, b1)


# speedup vs baseline: 3.2725x; 1.0053x over previous
import functools
import math

import jax
import jax.numpy as jnp
from jax import lax
from jax.experimental import pallas as pl
from jax.experimental.pallas import tpu as pltpu
from jax.experimental.pallas import tpu_sc as plsc

D_MODEL = 128
LANES = 16
NUM_CORES = 2
NUM_SUBCORES = 16
NUM_WORKERS = NUM_CORES * NUM_SUBCORES
SCALE = math.sqrt(D_MODEL)
NSTAGE = 4
LEAD = 3
JX = 4


@functools.partial(jax.jit, static_argnames=("b0", "b1"))
def _lookup(idx, table, b0, b1):
    mesh = plsc.VectorSubcoreMesh(core_axis_name="c", subcore_axis_name="s")
    xr = b0 // NUM_WORKERS
    nchunks = xr // 2
    njumbo = xr // JX
    cw = idx.shape[2]

    @functools.partial(
        pl.kernel,
        mesh=mesh,
        out_type=jax.ShapeDtypeStruct((b0, b1, D_MODEL), jnp.float32),
        scratch_types=[
            pltpu.VMEM((nchunks, cw), jnp.int32),
            pltpu.VMEM((NSTAGE, 2 * b1, D_MODEL), jnp.float32),
            pltpu.VMEM((2, JX, b1, D_MODEL), jnp.float32),
            pltpu.SemaphoreType.DMA((NSTAGE,)),
            pltpu.SemaphoreType.DMA((2,)),
        ],
    )
    def k(idx_hbm, table_hbm, out_hbm, idx_v, stag_v, jum_v, gsem, osem):
        cid = lax.axis_index("c")
        sid = lax.axis_index("s")
        wid = sid * NUM_CORES + cid
        x_base = wid * xr
        pltpu.sync_copy(idx_hbm.at[wid], idx_v)

        def start_gather(j, sb):
            pltpu.async_copy(
                table_hbm.at[idx_v.at[j, pl.ds(0, 2 * b1)]], stag_v.at[sb], gsem.at[sb]
            )

        def wait_gather(j, sb):
            pltpu.make_async_copy(
                table_hbm.at[idx_v.at[j, pl.ds(0, 2 * b1)]], stag_v.at[sb], gsem.at[sb]
            ).wait()

        def start_scatter(q, qb):
            pltpu.async_copy(
                jum_v.at[qb], out_hbm.at[pl.ds(x_base + q * JX, JX)], osem.at[qb]
            )

        def wait_scatter(qb):
            pltpu.make_async_copy(
                jum_v.at[qb], out_hbm.at[pl.ds(0, JX)], osem.at[qb]
            ).wait()

        for j in range(LEAD):
            start_gather(j, j)

        def outer(i, carry):
            j0 = i * 4
            q0 = i * 2
            for qoff in range(2):
                qb = qoff
                q = q0 + qoff
                for p in range(2):
                    jo = 2 * qoff + p
                    j = j0 + jo
                    sb = jo % NSTAGE

                    if p == 0:

                        @pl.when(q >= 2)
                        def _():
                            wait_scatter(qb)

                    @pl.when(j + LEAD < nchunks)
                    def _():
                        start_gather(j + LEAD, (jo + LEAD) % NSTAGE)

                    wait_gather(j, sb)

                    def row_body(r, c2):
                        for c in range(D_MODEL // LANES):
                            sl = pl.ds(c * LANES, LANES)
                            jum_v[qb, 2 * p, r, sl] = stag_v[sb, r, sl] * SCALE
                            jum_v[qb, 2 * p + 1, r, sl] = (
                                stag_v[sb, b1 + r, sl] * SCALE
                            )
                        return c2

                    lax.fori_loop(0, b1, row_body, 0, unroll=2)

                start_scatter(q, qb)
            return carry

        lax.fori_loop(0, nchunks // 4, outer, 0)

        for qb in range(2):
            wait_scatter(qb)

    return k(idx, table)


def kernel(x, table):
    b0, b1 = x.shape
    idx = x.astype(jnp.int32).reshape(b0 // 2, 2 * b1)
    cw = -(-(2 * b1) // 8) * 8
    if cw != 2 * b1:
        npad = cw - 2 * b1
        fill = (
            jnp.arange((b0 // 2) * npad, dtype=jnp.int32).reshape(b0 // 2, npad)
            * 9973 % jnp.int32(table.shape[0])
        )
        idx = jnp.concatenate([idx, fill], axis=1)
    idx = idx.reshape(NUM_WORKERS, b0 // (2 * NUM_WORKERS), cw)
    return _lookup(idx, table, b0, b1)


# speedup vs baseline: 3.2757x; 1.0010x over previous
import functools
import math

import jax
import jax.numpy as jnp
from jax import lax
from jax.experimental import pallas as pl
from jax.experimental.pallas import tpu as pltpu
from jax.experimental.pallas import tpu_sc as plsc

D_MODEL = 128
LANES = 16
NUM_CORES = 2
NUM_SUBCORES = 16
NUM_WORKERS = NUM_CORES * NUM_SUBCORES
SCALE = math.sqrt(D_MODEL)
NSTAGE = 4
LEAD = 3
JX = 4


@functools.partial(jax.jit, static_argnames=("b0", "b1"))
def _lookup(idx, table, b0, b1):
    mesh = plsc.VectorSubcoreMesh(core_axis_name="c", subcore_axis_name="s")
    xr = b0 // NUM_WORKERS
    nchunks = xr // 2
    njumbo = xr // JX
    cw = idx.shape[2]

    @functools.partial(
        pl.kernel,
        mesh=mesh,
        out_type=jax.ShapeDtypeStruct((b0, b1, D_MODEL), jnp.float32),
        scratch_types=[
            pltpu.VMEM((nchunks, cw), jnp.int32),
            pltpu.VMEM((NSTAGE, 2 * b1, D_MODEL), jnp.float32),
            pltpu.VMEM((2, JX, b1, D_MODEL), jnp.float32),
            pltpu.SemaphoreType.DMA((NSTAGE,)),
            pltpu.SemaphoreType.DMA((2,)),
        ],
    )
    def k(idx_hbm, table_hbm, out_hbm, idx_v, stag_v, jum_v, gsem, osem):
        cid = lax.axis_index("c")
        sid = lax.axis_index("s")
        wid = sid * NUM_CORES + cid
        x_base = wid * xr
        pltpu.sync_copy(idx_hbm.at[wid], idx_v)

        def start_gather(j, sb):
            pltpu.async_copy(
                table_hbm.at[idx_v.at[j, pl.ds(0, 2 * b1)]], stag_v.at[sb], gsem.at[sb]
            )

        def wait_gather(j, sb):
            pltpu.make_async_copy(
                table_hbm.at[idx_v.at[j, pl.ds(0, 2 * b1)]], stag_v.at[sb], gsem.at[sb]
            ).wait()

        def start_scatter(q, qb):
            pltpu.async_copy(
                jum_v.at[qb], out_hbm.at[pl.ds(x_base + q * JX, JX)], osem.at[qb]
            )

        def wait_scatter(qb):
            pltpu.make_async_copy(
                jum_v.at[qb], out_hbm.at[pl.ds(0, JX)], osem.at[qb]
            ).wait()

        for j in range(LEAD):
            start_gather(j, j)

        def outer(i, carry):
            j0 = i * 4
            q0 = i * 2
            for qoff in range(2):
                qb = qoff
                q = q0 + qoff
                for p in range(2):
                    jo = 2 * qoff + p
                    j = j0 + jo
                    sb = jo % NSTAGE

                    if p == 0:

                        @pl.when(q >= 2)
                        def _():
                            wait_scatter(qb)

                    @pl.when(j + LEAD < nchunks)
                    def _():
                        start_gather(j + LEAD, (jo + LEAD) % NSTAGE)

                    wait_gather(j, sb)

                    def row_body(r, c2):
                        for c in range(D_MODEL // LANES):
                            sl = pl.ds(c * LANES, LANES)
                            jum_v[qb, 2 * p, r, sl] = stag_v[sb, r, sl] * SCALE
                            jum_v[qb, 2 * p + 1, r, sl] = (
                                stag_v[sb, b1 + r, sl] * SCALE
                            )
                        return c2

                    lax.fori_loop(0, b1, row_body, 0, unroll=5)

                start_scatter(q, qb)
            return carry

        lax.fori_loop(0, nchunks // 4, outer, 0)

        for qb in range(2):
            wait_scatter(qb)

    return k(idx, table)


def kernel(x, table):
    b0, b1 = x.shape
    idx = x.astype(jnp.int32).reshape(b0 // 2, 2 * b1)
    cw = -(-(2 * b1) // 8) * 8
    if cw != 2 * b1:
        npad = cw - 2 * b1
        fill = (
            jnp.arange((b0 // 2) * npad, dtype=jnp.int32).reshape(b0 // 2, npad)
            * 9973 % jnp.int32(table.shape[0])
        )
        idx = jnp.concatenate([idx, fill], axis=1)
    idx = idx.reshape(NUM_WORKERS, b0 // (2 * NUM_WORKERS), cw)
    return _lookup(idx, table, b0, b1)


# speedup vs baseline: 3.6387x; 1.1108x over previous
import functools
import math

import jax
import jax.numpy as jnp
from jax import lax
from jax.experimental import pallas as pl
from jax.experimental.pallas import tpu as pltpu
from jax.experimental.pallas import tpu_sc as plsc

D_MODEL = 128
LANES = 16
NUM_CORES = 2
NUM_SUBCORES = 16
NUM_WORKERS = NUM_CORES * NUM_SUBCORES
SCALE = math.sqrt(D_MODEL)
NSTAGE = 4
LEAD = 3
JX = 4


@functools.partial(jax.jit, static_argnames=("b0", "b1"))
def _lookup(idx, table, b0, b1):
    mesh = plsc.VectorSubcoreMesh(core_axis_name="c", subcore_axis_name="s")
    xr = b0 // NUM_WORKERS
    nchunks = xr // 2
    njumbo = xr // JX
    cw = idx.shape[2]

    @functools.partial(
        pl.kernel,
        mesh=mesh,
        out_type=jax.ShapeDtypeStruct((b0, b1, D_MODEL), jnp.float32),
        scratch_types=[
            pltpu.VMEM((nchunks, cw), jnp.int32),
            pltpu.VMEM((NSTAGE, 2 * b1, D_MODEL), jnp.float32),
            pltpu.VMEM((2, JX, b1, D_MODEL), jnp.float32),
            pltpu.SemaphoreType.DMA((NSTAGE,)),
            pltpu.SemaphoreType.DMA((2,)),
        ],
    )
    def k(idx_hbm, table_hbm, out_hbm, idx_v, stag_v, jum_v, gsem, osem):
        cid = lax.axis_index("c")
        sid = lax.axis_index("s")
        wid = sid * NUM_CORES + cid
        x_base = wid * xr
        pltpu.sync_copy(idx_hbm.at[wid], idx_v)

        def start_gather(j, sb):
            pltpu.async_copy(
                table_hbm.at[idx_v.at[j, pl.ds(0, 2 * b1)]], stag_v.at[sb], gsem.at[sb]
            )

        def wait_gather(j, sb):
            pltpu.make_async_copy(
                table_hbm.at[idx_v.at[j, pl.ds(0, 2 * b1)]], stag_v.at[sb], gsem.at[sb]
            ).wait()

        def start_scatter(q, qb):
            pltpu.async_copy(
                jum_v.at[qb], out_hbm.at[pl.ds(x_base + q * JX, JX)], osem.at[qb]
            )

        def wait_scatter(qb):
            pltpu.make_async_copy(
                jum_v.at[qb], out_hbm.at[pl.ds(0, JX)], osem.at[qb]
            ).wait()

        for j in range(LEAD):
            start_gather(j, j)

        def outer(i, carry):
            j0 = i * 4
            q0 = i * 2
            for qoff in range(2):
                qb = qoff
                q = q0 + qoff
                for p in range(2):
                    jo = 2 * qoff + p
                    j = j0 + jo
                    sb = jo % NSTAGE

                    if p == 0:

                        @pl.when(q >= 2)
                        def _():
                            wait_scatter(qb)

                    @pl.when(j + LEAD < nchunks)
                    def _():
                        start_gather(j + LEAD, (jo + LEAD) % NSTAGE)

                    wait_gather(j, sb)

                    @plsc.parallel_loop(0, b1, unroll=2)
                    def row_body(r):
                        for c in range(D_MODEL // LANES):
                            sl = pl.ds(c * LANES, LANES)
                            jum_v[qb, 2 * p, r, sl] = stag_v[sb, r, sl] * SCALE
                            jum_v[qb, 2 * p + 1, r, sl] = (
                                stag_v[sb, b1 + r, sl] * SCALE
                            )

                start_scatter(q, qb)
            return carry

        lax.fori_loop(0, nchunks // 4, outer, 0)

        for qb in range(2):
            wait_scatter(qb)

    return k(idx, table)


def kernel(x, table):
    b0, b1 = x.shape
    idx = x.astype(jnp.int32).reshape(b0 // 2, 2 * b1)
    cw = -(-(2 * b1) // 8) * 8
    if cw != 2 * b1:
        npad = cw - 2 * b1
        fill = (
            jnp.arange((b0 // 2) * npad, dtype=jnp.int32).reshape(b0 // 2, npad)
            * 9973 % jnp.int32(table.shape[0])
        )
        idx = jnp.concatenate([idx, fill], axis=1)
    idx = idx.reshape(NUM_WORKERS, b0 // (2 * NUM_WORKERS), cw)
    return _lookup(idx, table, b0, b1)
